# Initial kernel scaffold; baseline (speedup 1.0000x reference)
import jax
import jax.numpy as jnp
from jax.experimental import pallas as pl


def kernel(x, edge_index, W0, b0, conv_w, bn_gamma, bn_beta, W1, b1):
    raise NotImplementedError("write your pallas kernel here")


# speedup vs baseline: 6.8409x; 6.8409x over previous
import functools
import math

import jax
import jax.numpy as jnp
from jax import lax
from jax.experimental import pallas as pl
from jax.experimental.pallas import tpu as pltpu
from jax.experimental.pallas import tpu_sc as plsc

N = 10000
E = 320000
IN_CH = 128
HID = 256
HALF = 128
OUT_CH = 40
L = 8
ALPHA = 0.1
THETA = 0.5
BN_EPS = 1e-5

NT = 16
NC = 2
CHUNK = 128
ROUND = NT * CHUNK
NCHUNK = -(-E // ROUND)
EPAD = NCHUNK * ROUND
EPT = EPAD // NT
NP = 10240
ZROWS = NP // NT

NB = 1024
GRID = NP // NB

_f32 = jnp.float32
_MESH = plsc.VectorSubcoreMesh(core_axis_name="c", subcore_axis_name="s")


@functools.partial(
    pl.kernel,
    out_type=jax.ShapeDtypeStruct((NC, NP, HALF), _f32),
    mesh=_MESH,
    scratch_types=[
        pltpu.VMEM_SHARED((NP, HALF), _f32),
        pltpu.VMEM((CHUNK, HALF), _f32),
        pltpu.VMEM((CHUNK,), jnp.int32),
        pltpu.VMEM((CHUNK,), jnp.int32),
        pltpu.SemaphoreType.DMA,
    ],
)
def _spmm_sc(gcat, rows2, cols, zeros, out, acc, gbuf, gidx, sidx, sem):
    c = lax.axis_index("c")
    s = lax.axis_index("s")
    pltpu.sync_copy(zeros.at[pl.ds(s * ZROWS, ZROWS)], acc.at[pl.ds(s * ZROWS, ZROWS)])
    plsc.subcore_barrier()
    base = c * EPAD + s * EPT

    def body(k, carry):
        off = base + k * CHUNK
        pltpu.sync_copy(rows2.at[pl.ds(off, CHUNK)], gidx)
        pltpu.sync_copy(cols.at[pl.ds(off - base + s * EPT, CHUNK)], sidx)
        pltpu.async_copy(gcat.at[gidx], gbuf, sem).wait()
        pltpu.sync_copy(gbuf, acc.at[sidx], add=True)
        return carry

    lax.fori_loop(0, NCHUNK, body, 0)
    plsc.subcore_barrier()
    pltpu.sync_copy(acc.at[pl.ds(s * ZROWS, ZROWS)],
                    out.at[c, pl.ds(s * ZROWS, ZROWS)])


@functools.partial(
    pl.kernel,
    out_type=jax.ShapeDtypeStruct((NC, NP, HALF), _f32),
    mesh=_MESH,
    scratch_types=[
        pltpu.VMEM_SHARED((NP, HALF), _f32),
        pltpu.VMEM((CHUNK, HALF), _f32),
        pltpu.VMEM((CHUNK,), jnp.int32),
        pltpu.SemaphoreType.DMA,
    ],
)
def _deg_sc(cols, onehot, zeros, out, acc, obuf, sidx, sem):
    c = lax.axis_index("c")
    s = lax.axis_index("s")
    pltpu.sync_copy(zeros.at[pl.ds(s * ZROWS, ZROWS)], acc.at[pl.ds(s * ZROWS, ZROWS)])
    pltpu.sync_copy(onehot, obuf)
    plsc.subcore_barrier()
    base = s * EPT

    def body(k, carry):
        off = base + k * CHUNK
        pltpu.sync_copy(cols.at[pl.ds(off, CHUNK)], sidx)
        pltpu.sync_copy(obuf, acc.at[sidx], add=True)
        return carry

    lax.fori_loop(0, NCHUNK, body, 0)
    plsc.subcore_barrier()
    pltpu.sync_copy(acc.at[pl.ds(s * ZROWS, ZROWS)],
                    out.at[c, pl.ds(s * ZROWS, ZROWS)])


def _pre_body(x_ref, w0_ref, b0_ref, deg_ref, x0s_ref, g2_ref, dinv_ref):
    x = x_ref[...]
    h = jnp.maximum(jnp.dot(x, w0_ref[...], preferred_element_type=_f32)
                    + b0_ref[0:1, :], 0.0)
    d = 1.0 + deg_ref[0][:, 0:1]
    dinv = lax.rsqrt(d)
    x0s_ref[...] = ALPHA * h
    g = dinv * h
    g2_ref[...] = jnp.stack([g[:, :HALF], g[:, HALF:]], axis=0)
    dinv_ref[...] = jnp.broadcast_to(dinv, (NB, HALF))


def _mix_t(s2_ref, g2_ref, x0s_ref, dinv_ref):
    dinv = dinv_ref[...]
    x0s = x0s_ref[...]
    t_lo = (1.0 - ALPHA) * dinv * (s2_ref[0] + g2_ref[0]) + x0s[:, :HALF]
    t_hi = (1.0 - ALPHA) * dinv * (s2_ref[1] + g2_ref[1]) + x0s[:, HALF:]
    return jnp.concatenate([t_lo, t_hi], axis=1)


def _layer_body(s2_ref, g2_ref, x0s_ref, dinv_ref, wp_ref, ac_ref, gout_ref):
    t = _mix_t(s2_ref, g2_ref, x0s_ref, dinv_ref)
    out = (t * ac_ref[0:1, :]
           + jnp.dot(t, wp_ref[...], preferred_element_type=_f32)
           + ac_ref[1:2, :])
    h = jnp.maximum(out, 0.0)
    dinv = dinv_ref[...]
    gout_ref[...] = jnp.stack([dinv * h[:, :HALF], dinv * h[:, HALF:]], axis=0)


def _final_body(s2_ref, g2_ref, x0s_ref, dinv_ref, wp_ref, ac_ref,
                w1_ref, b1_ref, y_ref):
    t = _mix_t(s2_ref, g2_ref, x0s_ref, dinv_ref)
    out = (t * ac_ref[0:1, :]
           + jnp.dot(t, wp_ref[...], preferred_element_type=_f32)
           + ac_ref[1:2, :])
    h = jnp.maximum(out, 0.0)
    y_ref[...] = (jnp.dot(h, w1_ref[...], preferred_element_type=_f32)
                  + b1_ref[0:1, :])


def _nb(i):
    return (i, 0)


_pre_tc = pl.pallas_call(
    _pre_body,
    grid=(GRID,),
    in_specs=[
        pl.BlockSpec((NB, IN_CH), _nb),
        pl.BlockSpec((IN_CH, HID), lambda i: (0, 0)),
        pl.BlockSpec((8, HID), lambda i: (0, 0)),
        pl.BlockSpec((NC, NB, HALF), lambda i: (0, i, 0)),
    ],
    out_specs=[
        pl.BlockSpec((NB, HID), _nb),
        pl.BlockSpec((NC, NB, HALF), lambda i: (0, i, 0)),
        pl.BlockSpec((NB, HALF), _nb),
    ],
    out_shape=[
        jax.ShapeDtypeStruct((NP, HID), _f32),
        jax.ShapeDtypeStruct((NC, NP, HALF), _f32),
        jax.ShapeDtypeStruct((NP, HALF), _f32),
    ],
)

_layer_tc = pl.pallas_call(
    _layer_body,
    grid=(GRID,),
    in_specs=[
        pl.BlockSpec((NC, NB, HALF), lambda i: (0, i, 0)),
        pl.BlockSpec((NC, NB, HALF), lambda i: (0, i, 0)),
        pl.BlockSpec((NB, HID), _nb),
        pl.BlockSpec((NB, HALF), _nb),
        pl.BlockSpec((HID, HID), lambda i: (0, 0)),
        pl.BlockSpec((8, HID), lambda i: (0, 0)),
    ],
    out_specs=[pl.BlockSpec((NC, NB, HALF), lambda i: (0, i, 0))],
    out_shape=[jax.ShapeDtypeStruct((NC, NP, HALF), _f32)],
)

_final_tc = pl.pallas_call(
    _final_body,
    grid=(GRID,),
    in_specs=[
        pl.BlockSpec((NC, NB, HALF), lambda i: (0, i, 0)),
        pl.BlockSpec((NC, NB, HALF), lambda i: (0, i, 0)),
        pl.BlockSpec((NB, HID), _nb),
        pl.BlockSpec((NB, HALF), _nb),
        pl.BlockSpec((HID, HID), lambda i: (0, 0)),
        pl.BlockSpec((8, HID), lambda i: (0, 0)),
        pl.BlockSpec((HID, OUT_CH), lambda i: (0, 0)),
        pl.BlockSpec((8, OUT_CH), lambda i: (0, 0)),
    ],
    out_specs=[pl.BlockSpec((NB, OUT_CH), _nb)],
    out_shape=[jax.ShapeDtypeStruct((NP, OUT_CH), _f32)],
)


def kernel(x, edge_index, W0, b0, conv_w, bn_gamma, bn_beta, W1, b1):
    rows = edge_index[0]
    cols = edge_index[1]
    pad = EPAD - E
    rows_p = jnp.concatenate([rows, jnp.zeros((pad,), jnp.int32)])
    rows2 = jnp.concatenate([rows_p, rows_p + NP])
    cols_p = jnp.concatenate([cols, jnp.full((pad,), N, jnp.int32)])
    zeros = jnp.zeros((NP, HALF), _f32)
    onehot = jnp.zeros((CHUNK, HALF), _f32).at[:, 0].set(1.0)
    xp = jnp.concatenate([x, jnp.zeros((NP - N, IN_CH), _f32)])

    deg2 = _deg_sc(cols_p, onehot, zeros)

    gh = bn_gamma * (1.0 / math.sqrt(1.0 + BN_EPS))
    betas = jnp.array([math.log(THETA / (l + 1) + 1.0) for l in range(L)], _f32)
    A = (1.0 - betas)[:, None] * gh
    Wp = conv_w * (betas[:, None] * gh)[:, None, :]
    ACs = jnp.zeros((L, 8, HID), _f32).at[:, 0].set(A).at[:, 1].set(bn_beta)
    b0c = jnp.zeros((8, HID), _f32).at[0].set(b0)
    b1c = jnp.zeros((8, OUT_CH), _f32).at[0].set(b1)

    x0s, g2, dinvb = _pre_tc(xp, W0, b0c, deg2)
    for l in range(L - 1):
        s2 = _spmm_sc(g2.reshape(NC * NP, HALF), rows2, cols_p, zeros)
        (g2,) = _layer_tc(s2, g2, x0s, dinvb, Wp[l], ACs[l])
    s2 = _spmm_sc(g2.reshape(NC * NP, HALF), rows2, cols_p, zeros)
    (y,) = _final_tc(s2, g2, x0s, dinvb, Wp[L - 1], ACs[L - 1], W1, b1c)
    return y[:N]
